# baseline (speedup 1.0000x reference)
import functools

import jax
import jax.numpy as jnp
from jax import lax
from jax.experimental import pallas as pl
from jax.experimental.pallas import tpu as pltpu
from jax.experimental.pallas import tpu_sc as plsc

_U_LO = 122 << 7
_U_HI = (131 << 7) - 1
_NLUT = _U_HI - _U_LO + 1

_B, _C, _H, _W = 8, 192, 64, 64
_N = _B * _C * _H * _W
_NC, _NS, _LANES = 2, 16, 16
_NW = _NC * _NS
_PER_W = _N // _NW
_CH = 16384
_NCHUNK = _PER_W // _CH
_NBUF = 2


def _build_luts(scale_table):
    st = scale_table[:63]
    keys = jnp.arange(_NLUT, dtype=jnp.int32) + _U_LO
    left = lax.bitcast_convert_type(keys << 16, jnp.float32)
    base = jnp.sum((st[None, :] < left[:, None]), axis=1).astype(jnp.int32)
    padded = jnp.concatenate([st, jnp.full((1,), jnp.inf, jnp.float32)])
    thr = padded[base]
    return base, thr


def _sc_bucketize(flat, base_lut, thr_lut):
    mesh = plsc.VectorSubcoreMesh(core_axis_name="c", subcore_axis_name="s")

    @functools.partial(
        pl.kernel,
        out_type=jax.ShapeDtypeStruct((_N,), jnp.int32),
        mesh=mesh,
        scratch_types=[
            pltpu.VMEM((_NLUT,), jnp.int32),
            pltpu.VMEM((_NLUT,), jnp.float32),
            pltpu.VMEM((_NBUF, _CH), jnp.float32),
            pltpu.VMEM((_NBUF, _CH), jnp.int32),
            pltpu.SemaphoreType.DMA((_NBUF,)),
            pltpu.SemaphoreType.DMA((_NBUF,)),
        ],
        compiler_params=pltpu.CompilerParams(needs_layout_passes=False),
    )
    def k(scales_hbm, base_hbm, thr_hbm, out_hbm,
          base_v, thr_v, in_v, out_v, in_sem, out_sem):
        wid = lax.axis_index("s") * _NC + lax.axis_index("c")
        pltpu.sync_copy(base_hbm, base_v)
        pltpu.sync_copy(thr_hbm, thr_v)
        base_off = wid * _PER_W

        def in_copy(ch, slot):
            return pltpu.make_async_copy(
                scales_hbm.at[pl.ds(base_off + ch * _CH, _CH)],
                in_v.at[slot], in_sem.at[slot])

        def out_copy(ch, slot):
            return pltpu.make_async_copy(
                out_v.at[slot],
                out_hbm.at[pl.ds(base_off + ch * _CH, _CH)],
                out_sem.at[slot])

        in_copy(0, 0).start()
        for ch in range(_NCHUNK):
            slot = ch % _NBUF
            if ch + 1 < _NCHUNK:
                in_copy(ch + 1, (ch + 1) % _NBUF).start()
            in_copy(ch, slot).wait()
            if ch >= _NBUF:
                out_copy(ch - _NBUF, slot).wait()

            @plsc.parallel_loop(0, _CH // _LANES, unroll=8)
            def body(i):
                s = in_v[slot, pl.ds(i * _LANES, _LANES)]
                u = (lax.bitcast_convert_type(s, jnp.int32) >> 16) - _U_LO
                u = jnp.minimum(jnp.maximum(u, 0), _NLUT - 1)
                b = plsc.load_gather(base_v, [u])
                t = plsc.load_gather(thr_v, [u])
                out_v[slot, pl.ds(i * _LANES, _LANES)] = jnp.where(s > t, b + 1, b)

            out_copy(ch, slot).start()
        for ch in range(_NCHUNK - _NBUF, _NCHUNK):
            out_copy(ch, ch % _NBUF).wait()

    return k(flat, base_lut, thr_lut)


def kernel(scales, scale_table):
    base_lut, thr_lut = _build_luts(scale_table)
    out = _sc_bucketize(scales.reshape(_N), base_lut, thr_lut)
    return out.reshape(scales.shape)


# speedup vs baseline: 1.1195x; 1.1195x over previous
import functools

import jax
import jax.numpy as jnp
from jax import lax
from jax.experimental import pallas as pl
from jax.experimental.pallas import tpu as pltpu
from jax.experimental.pallas import tpu_sc as plsc

_U_LO = 122 << 7
_U_HI = (131 << 7) - 1
_NLUT = _U_HI - _U_LO + 1

_B, _C, _H, _W = 8, 192, 64, 64
_HW = _H * _W
_NC, _NS, _LANES = 2, 16, 16
_NW = _NC * _NS
_C_PER_W = _B * _C // _NW
_SLAB = 3
_NCHUNK = _C_PER_W // _SLAB
_VREGS = _SLAB * _HW // _LANES
_NBUF = 2


def _build_luts(scale_table):
    st = scale_table[:63]
    keys = jnp.arange(_NLUT, dtype=jnp.int32) + _U_LO
    left = lax.bitcast_convert_type(keys << 16, jnp.float32)
    base = jnp.sum((st[None, :] < left[:, None]), axis=1).astype(jnp.int32)
    padded = jnp.concatenate([st, jnp.full((1,), jnp.inf, jnp.float32)])
    thr = padded[base]
    return base, thr


def _sc_bucketize(scales, base_lut, thr_lut):
    mesh = plsc.VectorSubcoreMesh(core_axis_name="c", subcore_axis_name="s")

    @functools.partial(
        pl.kernel,
        out_type=jax.ShapeDtypeStruct((_B, _C, _H, _W), jnp.int32),
        mesh=mesh,
        scratch_types=[
            pltpu.VMEM((_NLUT,), jnp.int32),
            pltpu.VMEM((_NLUT,), jnp.float32),
            pltpu.VMEM((_NBUF, _SLAB, _H, _W), jnp.float32),
            pltpu.VMEM((_NBUF, _SLAB, _H, _W), jnp.int32),
            pltpu.SemaphoreType.DMA((_NBUF,)),
            pltpu.SemaphoreType.DMA((_NBUF,)),
        ],
        compiler_params=pltpu.CompilerParams(needs_layout_passes=False),
    )
    def k(scales_hbm, base_hbm, thr_hbm, out_hbm,
          base_v, thr_v, in_v, out_v, in_sem, out_sem):
        wid = lax.axis_index("s") * _NC + lax.axis_index("c")
        pltpu.sync_copy(base_hbm, base_v)
        pltpu.sync_copy(thr_hbm, thr_v)
        b = wid // (_C // _C_PER_W)
        c0 = (wid % (_C // _C_PER_W)) * _C_PER_W

        def in_copy(ch, slot):
            return pltpu.make_async_copy(
                scales_hbm.at[b, pl.ds(c0 + ch * _SLAB, _SLAB)],
                in_v.at[slot], in_sem.at[slot])

        def out_copy(ch, slot):
            return pltpu.make_async_copy(
                out_v.at[slot],
                out_hbm.at[b, pl.ds(c0 + ch * _SLAB, _SLAB)],
                out_sem.at[slot])

        in_copy(0, 0).start()
        for ch in range(_NCHUNK):
            slot = ch % _NBUF
            if ch + 1 < _NCHUNK:
                in_copy(ch + 1, (ch + 1) % _NBUF).start()
            in_copy(ch, slot).wait()
            if ch >= _NBUF:
                out_copy(ch - _NBUF, slot).wait()

            @plsc.parallel_loop(0, _VREGS, unroll=8)
            def body(i):
                sl = i // (_HW // _LANES)
                r = i % (_HW // _LANES)
                h = r // (_W // _LANES)
                w0 = (r % (_W // _LANES)) * _LANES
                s = in_v[slot, sl, h, pl.ds(w0, _LANES)]
                u = (lax.bitcast_convert_type(s, jnp.int32) >> 16) - _U_LO
                u = jnp.minimum(jnp.maximum(u, 0), _NLUT - 1)
                bb = plsc.load_gather(base_v, [u])
                t = plsc.load_gather(thr_v, [u])
                out_v[slot, sl, h, pl.ds(w0, _LANES)] = jnp.where(s > t, bb + 1, bb)

            out_copy(ch, slot).start()
        for ch in range(_NCHUNK - _NBUF, _NCHUNK):
            out_copy(ch, ch % _NBUF).wait()

    return k(scales, base_lut, thr_lut)


def kernel(scales, scale_table):
    base_lut, thr_lut = _build_luts(scale_table)
    return _sc_bucketize(scales, base_lut, thr_lut)


# speedup vs baseline: 3.4673x; 3.0972x over previous
import functools
import math

import jax
import jax.numpy as jnp
from jax import lax
from jax.experimental import pallas as pl
from jax.experimental.pallas import tpu as pltpu
from jax.experimental.pallas import tpu_sc as plsc

_U_LO = 122 << 7
_U_HI = (131 << 7) - 1
_NLUT = _U_HI - _U_LO + 1

_B, _C, _H, _W = 8, 192, 64, 64
_ROWS = _B * _H
_NC, _NS, _LANES = 2, 16, 16
_NW = _NC * _NS
_R_PER_W = _ROWS // _NW
_CVR = _C // _LANES
_NBUF = 2

_D = math.log2(16.0 / 0.11) / 63
_A = 1.0 / (128.0 * _D)
_BC = (-127.0 - math.log2(0.11)) / _D


def _sc_bucketize(scales3, scale_table):
    mesh = plsc.VectorSubcoreMesh(core_axis_name="c", subcore_axis_name="s")

    @functools.partial(
        pl.kernel,
        out_type=jax.ShapeDtypeStruct((_ROWS, _W, _C), jnp.int32),
        mesh=mesh,
        scratch_types=[
            pltpu.VMEM((64,), jnp.float32),
            pltpu.VMEM((80,), jnp.float32),
            pltpu.VMEM((_NLUT,), jnp.int32),
            pltpu.VMEM((_NLUT,), jnp.float32),
            pltpu.VMEM((_NBUF, _W, _C), jnp.float32),
            pltpu.VMEM((_NBUF, _W, _C), jnp.int32),
            pltpu.SemaphoreType.DMA((_NBUF,)),
            pltpu.SemaphoreType.DMA((_NBUF,)),
        ],
        compiler_params=pltpu.CompilerParams(needs_layout_passes=False),
    )
    def k(scales_hbm, table_hbm, out_hbm,
          tbl_v, tpad_v, base_v, thr_v, in_v, out_v, in_sem, out_sem):
        wid = lax.axis_index("s") * _NC + lax.axis_index("c")
        row0 = wid * _R_PER_W

        def in_copy(ch, slot):
            return pltpu.make_async_copy(
                scales_hbm.at[row0 + ch], in_v.at[slot], in_sem.at[slot])

        def out_copy(ch, slot):
            return pltpu.make_async_copy(
                out_v.at[slot], out_hbm.at[row0 + ch], out_sem.at[slot])

        in_copy(0, 0).start()
        in_copy(1, 1).start()

        pltpu.sync_copy(table_hbm, tbl_v)
        lanes = lax.iota(jnp.int32, _LANES)
        for kv in range(80 // _LANES):
            j = kv * _LANES + lanes - 1
            g = jnp.minimum(jnp.maximum(j, 0), 63)
            v = plsc.load_gather(tbl_v, [g])
            v = jnp.where(j < 0, -jnp.inf, jnp.where(j >= 63, jnp.inf, v))
            tpad_v[pl.ds(kv * _LANES, _LANES)] = v

        @plsc.parallel_loop(0, _NLUT // _LANES, unroll=2)
        def lut_body(kv):
            u = kv * _LANES + lanes + _U_LO
            left = lax.bitcast_convert_type(u << 16, jnp.float32)
            est = u.astype(jnp.float32) * _A + _BC
            c = (est + 32.0).astype(jnp.int32) - 32
            c = jnp.minimum(jnp.maximum(c, -1), 63)
            t1 = plsc.load_gather(tpad_v, [c + 1])
            t2 = plsc.load_gather(tpad_v, [c + 2])
            b = c + jnp.where(left > t1, 1, 0) + jnp.where(left > t2, 1, 0)
            thr = plsc.load_gather(tpad_v, [b + 1])
            base_v[pl.ds(kv * _LANES, _LANES)] = b
            thr_v[pl.ds(kv * _LANES, _LANES)] = thr

        for ch in range(_R_PER_W):
            slot = ch % _NBUF
            in_copy(ch, slot).wait()
            if ch >= _NBUF:
                out_copy(ch - _NBUF, slot).wait()

            @plsc.parallel_loop(0, _W, unroll=2)
            def body(h):
                for j in range(_CVR):
                    s = in_v[slot, h, pl.ds(j * _LANES, _LANES)]
                    u = (lax.bitcast_convert_type(s, jnp.int32) >> 16) - _U_LO
                    u = jnp.minimum(jnp.maximum(u, 0), _NLUT - 1)
                    b = plsc.load_gather(base_v, [u])
                    t = plsc.load_gather(thr_v, [u])
                    out_v[slot, h, pl.ds(j * _LANES, _LANES)] = (
                        jnp.where(s > t, b + 1, b))

            out_copy(ch, slot).start()
            if ch + _NBUF < _R_PER_W:
                in_copy(ch + _NBUF, slot).start()
        for ch in range(_R_PER_W - _NBUF, _R_PER_W):
            out_copy(ch, ch % _NBUF).wait()

    return k(scales3, scale_table)


def kernel(scales, scale_table):
    x = scales.transpose(0, 2, 3, 1).reshape(_ROWS, _W, _C)
    out = _sc_bucketize(x, scale_table)
    return out.reshape(_B, _H, _W, _C).transpose(0, 3, 1, 2)


# speedup vs baseline: 4.1415x; 1.1944x over previous
import functools
import math

import jax
import jax.numpy as jnp
from jax import lax
from jax.experimental import pallas as pl
from jax.experimental.pallas import tpu as pltpu
from jax.experimental.pallas import tpu_sc as plsc

_U_LO = 122 << 7
_U_HI = (131 << 7) - 1
_NLUT = _U_HI - _U_LO + 1

_B, _C, _H, _W = 8, 192, 64, 64
_ROWS = _B * _H
_NC, _NS, _LANES = 2, 16, 16
_NW = _NC * _NS
_R_PER_W = _ROWS // _NW
_CVR = _C // _LANES
_NBUF = 2

_D = math.log2(16.0 / 0.11) / 63
_A = 1.0 / (128.0 * _D)
_BC = (-127.0 - math.log2(0.11)) / _D


def _sc_bucketize(scales3, scale_table):
    mesh = plsc.VectorSubcoreMesh(core_axis_name="c", subcore_axis_name="s")

    @functools.partial(
        pl.kernel,
        out_type=jax.ShapeDtypeStruct((_ROWS, _W, _C), jnp.int32),
        mesh=mesh,
        scratch_types=[
            pltpu.VMEM((64,), jnp.float32),
            pltpu.VMEM((80,), jnp.float32),
            pltpu.VMEM((_NLUT,), jnp.int32),
            pltpu.VMEM((_NLUT,), jnp.float32),
            pltpu.VMEM((_NBUF, _W, _C), jnp.float32),
            pltpu.VMEM((_NBUF, _W, _C), jnp.int32),
            pltpu.SemaphoreType.DMA((_NBUF,)),
            pltpu.SemaphoreType.DMA((_NBUF,)),
        ],
        compiler_params=pltpu.CompilerParams(needs_layout_passes=False),
    )
    def k(scales_hbm, table_hbm, out_hbm,
          tbl_v, tpad_v, base_v, thr_v, in_v, out_v, in_sem, out_sem):
        wid = lax.axis_index("s") * _NC + lax.axis_index("c")
        row0 = wid * _R_PER_W

        def in_copy(ch, slot):
            return pltpu.make_async_copy(
                scales_hbm.at[row0 + ch], in_v.at[slot], in_sem.at[slot])

        def out_copy(ch, slot):
            return pltpu.make_async_copy(
                out_v.at[slot], out_hbm.at[row0 + ch], out_sem.at[slot])

        in_copy(0, 0).start()
        in_copy(1, 1).start()

        pltpu.sync_copy(table_hbm, tbl_v)
        lanes = lax.iota(jnp.int32, _LANES)
        for kv in range(80 // _LANES):
            j = kv * _LANES + lanes - 1
            g = jnp.minimum(jnp.maximum(j, 0), 63)
            v = plsc.load_gather(tbl_v, [g])
            v = jnp.where(j < 0, -jnp.inf, jnp.where(j >= 63, jnp.inf, v))
            tpad_v[pl.ds(kv * _LANES, _LANES)] = v

        @plsc.parallel_loop(0, _NLUT // _LANES, unroll=2)
        def lut_body(kv):
            u = kv * _LANES + lanes + _U_LO
            left = lax.bitcast_convert_type(u << 16, jnp.float32)
            est = u.astype(jnp.float32) * _A + _BC
            c = (est + 32.0).astype(jnp.int32) - 32
            c = jnp.minimum(jnp.maximum(c, -1), 63)
            t1 = plsc.load_gather(tpad_v, [c + 1])
            t2 = plsc.load_gather(tpad_v, [c + 2])
            b = c + jnp.where(left > t1, 1, 0) + jnp.where(left > t2, 1, 0)
            thr = plsc.load_gather(tpad_v, [b + 1])
            base_v[pl.ds(kv * _LANES, _LANES)] = b
            thr_v[pl.ds(kv * _LANES, _LANES)] = thr

        @pl.loop(0, _R_PER_W, step=_NBUF)
        def chunk_loop(ch0):
            for b in range(_NBUF):
                ch = ch0 + b
                in_copy(ch, b).wait()

                @pl.when(ch0 >= _NBUF)
                def _():
                    out_copy(ch - _NBUF, b).wait()

                @plsc.parallel_loop(0, _W, unroll=2)
                def body(h):
                    for j in range(_CVR):
                        s = in_v[b, h, pl.ds(j * _LANES, _LANES)]
                        u = (lax.bitcast_convert_type(s, jnp.int32) >> 16) - _U_LO
                        u = jnp.minimum(jnp.maximum(u, 0), _NLUT - 1)
                        bb = plsc.load_gather(base_v, [u])
                        t = plsc.load_gather(thr_v, [u])
                        out_v[b, h, pl.ds(j * _LANES, _LANES)] = (
                            jnp.where(s > t, bb + 1, bb))

                out_copy(ch, b).start()

                @pl.when(ch0 + _NBUF < _R_PER_W)
                def _():
                    in_copy(ch + _NBUF, b).start()

        for ch in range(_R_PER_W - _NBUF, _R_PER_W):
            out_copy(ch, ch % _NBUF).wait()

    return k(scales3, scale_table)


def kernel(scales, scale_table):
    x = scales.transpose(0, 2, 3, 1).reshape(_ROWS, _W, _C)
    out = _sc_bucketize(x, scale_table)
    return out.reshape(_B, _H, _W, _C).transpose(0, 3, 1, 2)


# speedup vs baseline: 4.1463x; 1.0012x over previous
import functools
import math

import jax
import jax.numpy as jnp
from jax import lax
from jax.experimental import pallas as pl
from jax.experimental.pallas import tpu as pltpu
from jax.experimental.pallas import tpu_sc as plsc

_U_LO = 122 << 7
_U_HI = (131 << 7) - 1
_NLUT = _U_HI - _U_LO + 1

_B, _C, _H, _W = 8, 192, 64, 64
_ROWS = _B * _H
_NC, _NS, _LANES = 2, 16, 16
_NW = _NC * _NS
_R_PER_W = _ROWS // _NW
_CVR = _C // _LANES
_NBUF = 2

_D = math.log2(16.0 / 0.11) / 63
_A = 1.0 / (128.0 * _D)
_BC = (-127.0 - math.log2(0.11)) / _D


def _sc_bucketize(scales3, scale_table):
    mesh = plsc.VectorSubcoreMesh(core_axis_name="c", subcore_axis_name="s")

    @functools.partial(
        pl.kernel,
        out_type=jax.ShapeDtypeStruct((_ROWS, _W, _C), jnp.int32),
        mesh=mesh,
        scratch_types=[
            pltpu.VMEM((64,), jnp.float32),
            pltpu.VMEM((80,), jnp.float32),
            pltpu.VMEM((_NLUT,), jnp.int32),
            pltpu.VMEM((_NLUT,), jnp.float32),
            pltpu.VMEM((_NBUF, _W, _C), jnp.float32),
            pltpu.VMEM((_NBUF, _W, _C), jnp.int32),
            pltpu.SemaphoreType.DMA((_NBUF,)),
            pltpu.SemaphoreType.DMA((_NBUF,)),
        ],
        compiler_params=pltpu.CompilerParams(
            needs_layout_passes=False,
            skip_device_barrier=True,
            disable_bounds_checks=True,
        ),
    )
    def k(scales_hbm, table_hbm, out_hbm,
          tbl_v, tpad_v, base_v, thr_v, in_v, out_v, in_sem, out_sem):
        wid = lax.axis_index("s") * _NC + lax.axis_index("c")
        row0 = wid * _R_PER_W

        def in_copy(ch, slot):
            return pltpu.make_async_copy(
                scales_hbm.at[row0 + ch], in_v.at[slot], in_sem.at[slot])

        def out_copy(ch, slot):
            return pltpu.make_async_copy(
                out_v.at[slot], out_hbm.at[row0 + ch], out_sem.at[slot])

        in_copy(0, 0).start()
        in_copy(1, 1).start()

        pltpu.sync_copy(table_hbm, tbl_v)
        lanes = lax.iota(jnp.int32, _LANES)
        for kv in range(80 // _LANES):
            j = kv * _LANES + lanes - 1
            g = jnp.minimum(jnp.maximum(j, 0), 63)
            v = plsc.load_gather(tbl_v, [g])
            v = jnp.where(j < 0, -jnp.inf, jnp.where(j >= 63, jnp.inf, v))
            tpad_v[pl.ds(kv * _LANES, _LANES)] = v

        @plsc.parallel_loop(0, _NLUT // _LANES, unroll=2)
        def lut_body(kv):
            u = kv * _LANES + lanes + _U_LO
            left = lax.bitcast_convert_type(u << 16, jnp.float32)
            est = u.astype(jnp.float32) * _A + _BC
            c = (est + 32.0).astype(jnp.int32) - 32
            c = jnp.minimum(jnp.maximum(c, -1), 63)
            t1 = plsc.load_gather(tpad_v, [c + 1])
            t2 = plsc.load_gather(tpad_v, [c + 2])
            b = c + jnp.where(left > t1, 1, 0) + jnp.where(left > t2, 1, 0)
            thr = plsc.load_gather(tpad_v, [b + 1])
            base_v[pl.ds(kv * _LANES, _LANES)] = b
            thr_v[pl.ds(kv * _LANES, _LANES)] = thr

        @pl.loop(0, _R_PER_W, step=_NBUF)
        def chunk_loop(ch0):
            for b in range(_NBUF):
                ch = ch0 + b
                in_copy(ch, b).wait()

                @pl.when(ch0 >= _NBUF)
                def _():
                    out_copy(ch - _NBUF, b).wait()

                @plsc.parallel_loop(0, _W, unroll=2)
                def body(h):
                    for j in range(_CVR):
                        s = in_v[b, h, pl.ds(j * _LANES, _LANES)]
                        u = (lax.bitcast_convert_type(s, jnp.int32) >> 16) - _U_LO
                        u = jnp.minimum(jnp.maximum(u, 0), _NLUT - 1)
                        bb = plsc.load_gather(base_v, [u])
                        t = plsc.load_gather(thr_v, [u])
                        out_v[b, h, pl.ds(j * _LANES, _LANES)] = (
                            jnp.where(s > t, bb + 1, bb))

                out_copy(ch, b).start()

                @pl.when(ch0 + _NBUF < _R_PER_W)
                def _():
                    in_copy(ch + _NBUF, b).start()

        for ch in range(_R_PER_W - _NBUF, _R_PER_W):
            out_copy(ch, ch % _NBUF).wait()

    return k(scales3, scale_table)


def kernel(scales, scale_table):
    x = scales.transpose(0, 2, 3, 1).reshape(_ROWS, _W, _C)
    out = _sc_bucketize(x, scale_table)
    return out.reshape(_B, _H, _W, _C).transpose(0, 3, 1, 2)
